# Initial kernel scaffold; baseline (speedup 1.0000x reference)
import jax
import jax.numpy as jnp
from jax.experimental import pallas as pl


def kernel(x, offsets, pe):
    raise NotImplementedError("write your pallas kernel here")


# speedup vs baseline: 2.0716x; 2.0716x over previous
import jax
import jax.numpy as jnp
from jax import lax
from jax.experimental import pallas as pl
from jax.experimental.pallas import tpu as pltpu
from jax.experimental.pallas import tpu_sc as plsc

S = 4096
B = 4
E = 2048
MAX_LEN = 8192
L = 16
NW = 32
SROWS_PER_W = S // NW
CHS = 2
FR = CHS * B
NCHUNK = SROWS_PER_W // CHS
NXB = 4
NPB = 3
SCALE = 512.0


def _body(x_hbm, off_hbm, pe_hbm, out_hbm,
          offv, xb0, xb1, xb2, xb3, pb0, pb1, pb2, ib0, ib1, ib2,
          sgx0, sgx1, sgx2, sgx3, sgp0, sgp1, sgp2, ssc0, ssc1, ssc2, ssc3):
    cid = lax.axis_index("c")
    sid = lax.axis_index("s")
    wid = sid * 2 + cid
    base_f = wid * SROWS_PER_W * B
    base_s = wid * SROWS_PER_W

    xbs = [xb0, xb1, xb2, xb3]
    pbs = [pb0, pb1, pb2]
    ibs = [ib0, ib1, ib2]
    sgx = [sgx0, sgx1, sgx2, sgx3]
    sgp = [sgp0, sgp1, sgp2]
    ssc = [ssc0, ssc1, ssc2, ssc3]

    lanes = lax.broadcasted_iota(jnp.int32, (L,), 0)

    offv[...] = jnp.zeros((L,), jnp.float32)
    pltpu.sync_copy(off_hbm, offv.at[pl.ds(0, B)])
    offi = (offv[...] * SCALE).astype(jnp.int32)
    dnums = lax.GatherDimensionNumbers(
        offset_dims=(), collapsed_slice_dims=(0,), start_index_map=(0,))
    offb_lane = lax.gather(offi, (lanes & (B - 1))[:, None], dnums, (1,),
                           mode=lax.GatherScatterMode.PROMISE_IN_BOUNDS)

    def fill_idx(c, m):
        s_lane = base_s + c * CHS + (lanes >> 2)
        pos = offb_lane + s_lane - 1
        ibs[m][...] = jnp.maximum(pos, 0)

    def x_src(c):
        return x_hbm.at[pl.ds(base_f + c * FR, FR)]

    def out_dst(c):
        return out_hbm.at[pl.ds(base_f + c * FR, FR)]

    def start_pg(c, m):
        fill_idx(c, m)
        pltpu.async_copy(pe_hbm.at[ibs[m].at[pl.ds(0, FR)]], pbs[m], sgp[m])

    def wait_pg(m):
        pltpu.make_async_copy(pe_hbm.at[ibs[m].at[pl.ds(0, FR)]],
                              pbs[m], sgp[m]).wait()

    pltpu.async_copy(x_src(0), xbs[0], sgx[0])
    start_pg(0, 0)
    pltpu.async_copy(x_src(1), xbs[1], sgx[1])
    start_pg(1, 1)

    for c in range(NCHUNK):
        kx, kp = c % NXB, c % NPB
        pltpu.make_async_copy(x_src(c), xbs[kx], sgx[kx]).wait()
        wait_pg(kp)

        if c == 0:
            lo = jnp.where(wid == 0, B * E, 0)
        else:
            lo = 0

        @plsc.parallel_loop(lo, FR * E, step=L, unroll=16)
        def _(i, kx=kx, kp=kp):
            r = i >> 11
            col = i & (E - 1)
            v = pbs[kp][r, pl.ds(col, L)]
            plsc.addupdate(xbs[kx].at[r, pl.ds(col, L)], v)

        pltpu.async_copy(xbs[kx], out_dst(c), ssc[kx])

        nc = c + 2
        if nc < NCHUNK:
            kx2, kp2 = nc % NXB, nc % NPB
            if nc >= NXB:
                pltpu.make_async_copy(xbs[kx2], out_dst(nc - NXB),
                                      ssc[kx2]).wait()
            pltpu.async_copy(x_src(nc), xbs[kx2], sgx[kx2])
            start_pg(nc, kp2)

    for m in range(NXB):
        if NCHUNK - NXB + m >= 0:
            pltpu.make_async_copy(xbs[(NCHUNK - NXB + m) % NXB],
                                  out_dst(0), ssc[(NCHUNK - NXB + m) % NXB]).wait()


@jax.jit
def _pe_add(xf, offsets, pe2d):
    mesh = plsc.VectorSubcoreMesh(core_axis_name="c", subcore_axis_name="s")
    f = pl.kernel(
        _body,
        out_type=jax.ShapeDtypeStruct((S * B, E), jnp.float32),
        mesh=mesh,
        compiler_params=pltpu.CompilerParams(use_tc_tiling_on_sc=False),
        scratch_types=[
            pltpu.VMEM((L,), jnp.float32),
            pltpu.VMEM((FR, E), jnp.float32),
            pltpu.VMEM((FR, E), jnp.float32),
            pltpu.VMEM((FR, E), jnp.float32),
            pltpu.VMEM((FR, E), jnp.float32),
            pltpu.VMEM((FR, E), jnp.float32),
            pltpu.VMEM((FR, E), jnp.float32),
            pltpu.VMEM((FR, E), jnp.float32),
            pltpu.VMEM((L,), jnp.int32),
            pltpu.VMEM((L,), jnp.int32),
            pltpu.VMEM((L,), jnp.int32),
            pltpu.SemaphoreType.DMA,
            pltpu.SemaphoreType.DMA,
            pltpu.SemaphoreType.DMA,
            pltpu.SemaphoreType.DMA,
            pltpu.SemaphoreType.DMA,
            pltpu.SemaphoreType.DMA,
            pltpu.SemaphoreType.DMA,
            pltpu.SemaphoreType.DMA,
            pltpu.SemaphoreType.DMA,
            pltpu.SemaphoreType.DMA,
            pltpu.SemaphoreType.DMA,
        ],
    )
    return f(xf, offsets, pe2d)


def kernel(x, offsets, pe):
    out = _pe_add(x.reshape(S * B, E), offsets, pe.reshape(MAX_LEN, E))
    return out.reshape(S, B, E)


# speedup vs baseline: 5.0182x; 2.4224x over previous
import jax
import jax.numpy as jnp
from jax import lax
from jax.experimental import pallas as pl
from jax.experimental.pallas import tpu as pltpu
from jax.experimental.pallas import tpu_sc as plsc

S = 4096
B = 4
E = 2048
MAX_LEN = 8192
L = 16
NW = 32
CB = E // 128
SEG = 128
SEGS_PER_S = CB * B
SROWS_PER_W = S // NW
CHS = 2
FR = CHS * SEGS_PER_S
NCHUNK = SROWS_PER_W // CHS
NG = FR // 128
SCALE = 512.0


def _body(x_hbm, off_hbm, pe_hbm, out_hbm,
          offv, xb0, xb1, xb2, pb0, pb1, ib0, ib1,
          sgx0, sgx1, sgx2, sgp0, sgp1, ssc0, ssc1, ssc2):
    cid = lax.axis_index("c")
    sid = lax.axis_index("s")
    wid = sid * 2 + cid
    base_g = wid * SROWS_PER_W * SEGS_PER_S
    base_s = wid * SROWS_PER_W

    xbs = [xb0, xb1, xb2]
    pbs = [pb0, pb1]
    ibs = [ib0, ib1]
    sgx = [sgx0, sgx1, sgx2]
    sgp = [sgp0, sgp1]
    ssc = [ssc0, ssc1, ssc2]

    lanes = lax.broadcasted_iota(jnp.int32, (L,), 0)

    offv[...] = jnp.zeros((L,), jnp.float32)
    pltpu.sync_copy(off_hbm, offv.at[pl.ds(0, B)])
    offi = (offv[...] * SCALE).astype(jnp.int32)
    dnums = lax.GatherDimensionNumbers(
        offset_dims=(), collapsed_slice_dims=(0,), start_index_map=(0,))
    offb_lane = lax.gather(offi, (lanes & (B - 1))[:, None], dnums, (1,),
                           mode=lax.GatherScatterMode.PROMISE_IN_BOUNDS)
    lane_cb4 = (lanes >> 2)

    def fill_idx(c, m):
        for g in range(FR // L):
            s_prime = g >> 2
            s_seq = base_s + c * CHS + s_prime
            cb = (4 * g + lane_cb4) & (CB - 1)
            pos = offb_lane + (s_seq - 1)
            idx = jnp.maximum(pos, 0) * CB + cb
            ibs[m][g >> 3, pl.ds((g & 7) * L, L)] = idx

    def x_src(c):
        return x_hbm.at[pl.ds(base_g + c * FR, FR)]

    def out_dst(c):
        return out_hbm.at[pl.ds(base_g + c * FR, FR)]

    def start_pg(c, m):
        fill_idx(c, m)
        for g in range(NG):
            pltpu.async_copy(pe_hbm.at[ibs[m].at[g]],
                             pbs[m].at[pl.ds(g * 128, 128)], sgp[m])

    def wait_pg(m):
        for g in range(NG):
            pltpu.make_async_copy(pe_hbm.at[ibs[m].at[g]],
                                  pbs[m].at[pl.ds(g * 128, 128)],
                                  sgp[m]).wait()

    pltpu.async_copy(x_src(0), xbs[0], sgx[0])
    start_pg(0, 0)
    pltpu.async_copy(x_src(1), xbs[1], sgx[1])
    start_pg(1, 1)

    for c in range(NCHUNK):
        kx, kp = c % 3, c % 2
        pltpu.make_async_copy(x_src(c), xbs[kx], sgx[kx]).wait()
        wait_pg(kp)

        if c == 0:
            lo = jnp.where(wid == 0, SEGS_PER_S * SEG, 0)
        else:
            lo = 0

        @plsc.parallel_loop(lo, FR * SEG, step=L, unroll=16)
        def _(i, kx=kx, kp=kp):
            r = i >> 7
            col = i & (SEG - 1)
            v = pbs[kp][r, pl.ds(col, L)]
            plsc.addupdate(xbs[kx].at[r, pl.ds(col, L)], v)

        pltpu.async_copy(xbs[kx], out_dst(c), ssc[kx])

        nc = c + 2
        if nc < NCHUNK:
            kx2, kp2 = nc % 3, nc % 2
            if nc >= 3:
                pltpu.make_async_copy(xbs[kx2], out_dst(nc - 3),
                                      ssc[kx2]).wait()
            pltpu.async_copy(x_src(nc), xbs[kx2], sgx[kx2])
            start_pg(nc, kp2)

    for m in range(3):
        pltpu.make_async_copy(xbs[m], out_dst(0), ssc[m]).wait()


@jax.jit
def _pe_add(xf, offsets, pef):
    mesh = plsc.VectorSubcoreMesh(core_axis_name="c", subcore_axis_name="s")
    f = pl.kernel(
        _body,
        out_type=jax.ShapeDtypeStruct((S * CB * B, SEG), jnp.float32),
        mesh=mesh,
        compiler_params=pltpu.CompilerParams(use_tc_tiling_on_sc=False),
        scratch_types=[
            pltpu.VMEM((L,), jnp.float32),
            pltpu.VMEM((FR, SEG), jnp.float32),
            pltpu.VMEM((FR, SEG), jnp.float32),
            pltpu.VMEM((FR, SEG), jnp.float32),
            pltpu.VMEM((FR, SEG), jnp.float32),
            pltpu.VMEM((FR, SEG), jnp.float32),
            pltpu.VMEM((NG, 128), jnp.int32),
            pltpu.VMEM((NG, 128), jnp.int32),
            pltpu.SemaphoreType.DMA,
            pltpu.SemaphoreType.DMA,
            pltpu.SemaphoreType.DMA,
            pltpu.SemaphoreType.DMA,
            pltpu.SemaphoreType.DMA,
            pltpu.SemaphoreType.DMA,
            pltpu.SemaphoreType.DMA,
            pltpu.SemaphoreType.DMA,
        ],
    )
    return f(xf, offsets, pef)


def kernel(x, offsets, pe):
    xf = x.reshape(S, B, CB, SEG).transpose(0, 2, 1, 3).reshape(S * CB * B, SEG)
    pef = pe.reshape(MAX_LEN * CB, SEG)
    out = _pe_add(xf, offsets, pef)
    out = out.reshape(S, CB, B, SEG).transpose(0, 2, 1, 3).reshape(S, B, E)
    return out


# speedup vs baseline: 5.0212x; 1.0006x over previous
import jax
import jax.numpy as jnp
from jax import lax
from jax.experimental import pallas as pl
from jax.experimental.pallas import tpu as pltpu
from jax.experimental.pallas import tpu_sc as plsc

S = 4096
B = 4
E = 2048
MAX_LEN = 8192
L = 16
NW = 32
CB = E // 128
SEG = 128
SEGS_PER_S = CB * B
SROWS_PER_W = S // NW
CHS = 2
FR = CHS * SEGS_PER_S
NCHUNK = SROWS_PER_W // CHS
NG = FR // 128
SCALE = 512.0


def _body(x_hbm, off_hbm, pe_hbm, out_hbm,
          offv, xb0, xb1, xb2, pb0, pb1, ib0, ib1, ib2, ib3,
          sgx0, sgx1, sgx2, sgp0, sgp1, ssc0, ssc1, ssc2):
    cid = lax.axis_index("c")
    sid = lax.axis_index("s")
    wid = sid * 2 + cid
    base_g = wid * SROWS_PER_W * SEGS_PER_S
    base_s = wid * SROWS_PER_W

    xbs = [xb0, xb1, xb2]
    pbs = [pb0, pb1]
    ibs = [ib0, ib1, ib2, ib3]
    sgx = [sgx0, sgx1, sgx2]
    sgp = [sgp0, sgp1]
    ssc = [ssc0, ssc1, ssc2]

    lanes = lax.broadcasted_iota(jnp.int32, (L,), 0)

    offv[...] = jnp.zeros((L,), jnp.float32)
    pltpu.sync_copy(off_hbm, offv.at[pl.ds(0, B)])
    offi = (offv[...] * SCALE).astype(jnp.int32)
    dnums = lax.GatherDimensionNumbers(
        offset_dims=(), collapsed_slice_dims=(0,), start_index_map=(0,))
    offb_lane = lax.gather(offi, (lanes & (B - 1))[:, None], dnums, (1,),
                           mode=lax.GatherScatterMode.PROMISE_IN_BOUNDS)
    lane_cb4 = (lanes >> 2)

    def fill_idx(c):
        m = c % len(ibs)
        for g in range(FR // L):
            s_prime = g >> 2
            s_seq = base_s + c * CHS + s_prime
            cb = (4 * g + lane_cb4) & (CB - 1)
            pos = offb_lane + (s_seq - 1)
            idx = jnp.maximum(pos, 0) * CB + cb
            ibs[m][pl.ds(g * L, L)] = idx

    def x_src(c):
        return x_hbm.at[pl.ds(base_g + c * FR, FR)]

    def out_dst(c):
        return out_hbm.at[pl.ds(base_g + c * FR, FR)]

    def start_pg(c, m):
        pltpu.async_copy(pe_hbm.at[ibs[c % len(ibs)]], pbs[m], sgp[m])

    def wait_pg(c, m):
        pltpu.make_async_copy(pe_hbm.at[ibs[c % len(ibs)]],
                              pbs[m], sgp[m]).wait()

    fill_idx(0)
    fill_idx(1)
    fill_idx(2)
    pltpu.async_copy(x_src(0), xbs[0], sgx[0])
    start_pg(0, 0)
    pltpu.async_copy(x_src(1), xbs[1], sgx[1])
    start_pg(1, 1)

    for c in range(NCHUNK):
        kx, kp = c % 3, c % 2
        if c + 3 < NCHUNK:
            fill_idx(c + 3)
        pltpu.make_async_copy(x_src(c), xbs[kx], sgx[kx]).wait()
        wait_pg(c, kp)

        if c == 0:
            lo = jnp.where(wid == 0, SEGS_PER_S * SEG, 0)
        else:
            lo = 0

        @plsc.parallel_loop(lo, FR * SEG, step=L, unroll=16)
        def _(i, kx=kx, kp=kp):
            r = i >> 7
            col = i & (SEG - 1)
            v = pbs[kp][r, pl.ds(col, L)]
            plsc.addupdate(xbs[kx].at[r, pl.ds(col, L)], v)

        pltpu.async_copy(xbs[kx], out_dst(c), ssc[kx])

        nc = c + 2
        if nc < NCHUNK:
            kx2, kp2 = nc % 3, nc % 2
            if nc >= 3:
                pltpu.make_async_copy(xbs[kx2], out_dst(nc - 3),
                                      ssc[kx2]).wait()
            pltpu.async_copy(x_src(nc), xbs[kx2], sgx[kx2])
            start_pg(nc, kp2)

    for m in range(3):
        pltpu.make_async_copy(xbs[m], out_dst(0), ssc[m]).wait()


@jax.jit
def _pe_add(xf, offsets, pef):
    mesh = plsc.VectorSubcoreMesh(core_axis_name="c", subcore_axis_name="s")
    f = pl.kernel(
        _body,
        out_type=jax.ShapeDtypeStruct((S * CB * B, SEG), jnp.float32),
        mesh=mesh,
        compiler_params=pltpu.CompilerParams(use_tc_tiling_on_sc=False),
        scratch_types=[
            pltpu.VMEM((L,), jnp.float32),
            pltpu.VMEM((FR, SEG), jnp.float32),
            pltpu.VMEM((FR, SEG), jnp.float32),
            pltpu.VMEM((FR, SEG), jnp.float32),
            pltpu.VMEM((FR, SEG), jnp.float32),
            pltpu.VMEM((FR, SEG), jnp.float32),
            pltpu.VMEM((FR,), jnp.int32),
            pltpu.VMEM((FR,), jnp.int32),
            pltpu.VMEM((FR,), jnp.int32),
            pltpu.VMEM((FR,), jnp.int32),
            pltpu.SemaphoreType.DMA,
            pltpu.SemaphoreType.DMA,
            pltpu.SemaphoreType.DMA,
            pltpu.SemaphoreType.DMA,
            pltpu.SemaphoreType.DMA,
            pltpu.SemaphoreType.DMA,
            pltpu.SemaphoreType.DMA,
            pltpu.SemaphoreType.DMA,
        ],
    )
    return f(xf, offsets, pef)


def kernel(x, offsets, pe):
    xf = x.reshape(S, B, CB, SEG).transpose(0, 2, 1, 3).reshape(S * CB * B, SEG)
    pef = pe.reshape(MAX_LEN * CB, SEG)
    out = _pe_add(xf, offsets, pef)
    out = out.reshape(S, CB, B, SEG).transpose(0, 2, 1, 3).reshape(S, B, E)
    return out


# speedup vs baseline: 5.0239x; 1.0005x over previous
import jax
import jax.numpy as jnp
from jax import lax
from jax.experimental import pallas as pl
from jax.experimental.pallas import tpu as pltpu
from jax.experimental.pallas import tpu_sc as plsc

S = 4096
B = 4
E = 2048
MAX_LEN = 8192
L = 16
NW = 32
CB = E // 128
SEG = 128
SEGS_PER_S = CB * B
SROWS_PER_W = S // NW
CHS = 2
FR = CHS * SEGS_PER_S
NCHUNK = SROWS_PER_W // CHS
NG = FR // 128
SCALE = 512.0


def _body(x_hbm, off_hbm, pe_hbm, out_hbm,
          offv, xb0, xb1, xb2, pb0, pb1, ib0, ib1, ib2, ib3,
          sgx0, sgx1, sgx2, sgp0, sgp1, ssc0, ssc1, ssc2):
    cid = lax.axis_index("c")
    sid = lax.axis_index("s")
    wid = sid * 2 + cid
    base_g = wid * SROWS_PER_W * SEGS_PER_S
    base_s = wid * SROWS_PER_W

    xbs = [xb0, xb1, xb2]
    pbs = [pb0, pb1]
    ibs = [ib0, ib1, ib2, ib3]
    sgx = [sgx0, sgx1, sgx2]
    sgp = [sgp0, sgp1]
    ssc = [ssc0, ssc1, ssc2]

    lanes = lax.broadcasted_iota(jnp.int32, (L,), 0)

    offv[...] = jnp.zeros((L,), jnp.float32)
    pltpu.sync_copy(off_hbm, offv.at[pl.ds(0, B)])
    offi = (offv[...] * SCALE).astype(jnp.int32)
    dnums = lax.GatherDimensionNumbers(
        offset_dims=(), collapsed_slice_dims=(0,), start_index_map=(0,))
    offb_lane = lax.gather(offi, (lanes & (B - 1))[:, None], dnums, (1,),
                           mode=lax.GatherScatterMode.PROMISE_IN_BOUNDS)
    lane_cb4 = (lanes >> 2)

    def fill_idx(c):
        m = c % len(ibs)
        for g in range(FR // L):
            s_prime = g >> 2
            s_seq = base_s + c * CHS + s_prime
            cb = (4 * g + lane_cb4) & (CB - 1)
            pos = offb_lane + (s_seq - 1)
            idx = jnp.maximum(pos, 0) * CB + cb
            ibs[m][pl.ds(g * L, L)] = idx

    def x_src(c):
        return x_hbm.at[pl.ds(base_g + c * FR, FR)]

    def out_dst(c):
        return out_hbm.at[pl.ds(base_g + c * FR, FR)]

    def start_pg(c, m):
        pltpu.async_copy(pe_hbm.at[ibs[c % len(ibs)]], pbs[m], sgp[m])

    def wait_pg(c, m):
        pltpu.make_async_copy(pe_hbm.at[ibs[c % len(ibs)]],
                              pbs[m], sgp[m]).wait()

    fill_idx(0)
    fill_idx(1)
    fill_idx(2)
    pltpu.async_copy(x_src(0), xbs[0], sgx[0])
    start_pg(0, 0)
    pltpu.async_copy(x_src(1), xbs[1], sgx[1])
    start_pg(1, 1)

    for c in range(NCHUNK):
        kx, kp = c % 3, c % 2
        if c + 3 < NCHUNK:
            fill_idx(c + 3)
        pltpu.make_async_copy(x_src(c), xbs[kx], sgx[kx]).wait()
        wait_pg(c, kp)

        if c == 0:
            lo = jnp.where(wid == 0, SEGS_PER_S, 0)
        else:
            lo = 0

        @plsc.parallel_loop(lo, FR, step=1, unroll=2)
        def _(r, kx=kx, kp=kp):
            for u in range(SEG // L):
                col = u * L
                v = pbs[kp][r, pl.ds(col, L)]
                plsc.addupdate(xbs[kx].at[r, pl.ds(col, L)], v)

        pltpu.async_copy(xbs[kx], out_dst(c), ssc[kx])

        nc = c + 2
        if nc < NCHUNK:
            kx2, kp2 = nc % 3, nc % 2
            if nc >= 3:
                pltpu.make_async_copy(xbs[kx2], out_dst(nc - 3),
                                      ssc[kx2]).wait()
            pltpu.async_copy(x_src(nc), xbs[kx2], sgx[kx2])
            start_pg(nc, kp2)

    for m in range(3):
        pltpu.make_async_copy(xbs[m], out_dst(0), ssc[m]).wait()


@jax.jit
def _pe_add(xf, offsets, pef):
    mesh = plsc.VectorSubcoreMesh(core_axis_name="c", subcore_axis_name="s")
    f = pl.kernel(
        _body,
        out_type=jax.ShapeDtypeStruct((S * CB * B, SEG), jnp.float32),
        mesh=mesh,
        compiler_params=pltpu.CompilerParams(use_tc_tiling_on_sc=False),
        scratch_types=[
            pltpu.VMEM((L,), jnp.float32),
            pltpu.VMEM((FR, SEG), jnp.float32),
            pltpu.VMEM((FR, SEG), jnp.float32),
            pltpu.VMEM((FR, SEG), jnp.float32),
            pltpu.VMEM((FR, SEG), jnp.float32),
            pltpu.VMEM((FR, SEG), jnp.float32),
            pltpu.VMEM((FR,), jnp.int32),
            pltpu.VMEM((FR,), jnp.int32),
            pltpu.VMEM((FR,), jnp.int32),
            pltpu.VMEM((FR,), jnp.int32),
            pltpu.SemaphoreType.DMA,
            pltpu.SemaphoreType.DMA,
            pltpu.SemaphoreType.DMA,
            pltpu.SemaphoreType.DMA,
            pltpu.SemaphoreType.DMA,
            pltpu.SemaphoreType.DMA,
            pltpu.SemaphoreType.DMA,
            pltpu.SemaphoreType.DMA,
        ],
    )
    return f(xf, offsets, pef)


def kernel(x, offsets, pe):
    xf = x.reshape(S, B, CB, SEG).transpose(0, 2, 1, 3).reshape(S * CB * B, SEG)
    pef = pe.reshape(MAX_LEN * CB, SEG)
    out = _pe_add(xf, offsets, pef)
    out = out.reshape(S, CB, B, SEG).transpose(0, 2, 1, 3).reshape(S, B, E)
    return out
